# initial kernel scaffold (unmeasured)
import jax
import jax.numpy as jnp
from jax import lax
from jax.experimental import pallas as pl
from jax.experimental.pallas import tpu as pltpu


def kernel(
    x,
):
    def body(*refs):
        pass

    out_shape = jax.ShapeDtypeStruct(..., jnp.float32)
    return pl.pallas_call(body, out_shape=out_shape)(...)


# baseline (device time: 2210781 ns/iter reference)
import jax

jax.config.update("jax_compilation_cache_dir", "/tmp/scband_jax_cache")
jax.config.update("jax_persistent_cache_min_compile_time_secs", 0.0)
jax.config.update("jax_persistent_cache_min_entry_size_bytes", 0)

import jax.numpy as jnp
from jax import lax
from jax.experimental import pallas as pl
from jax.experimental.pallas import tpu as pltpu

N_DEV = 4
M_LOC = 8192
N = 1024
M = N_DEV * M_LOC
NCOL = N // N_DEV
B = 128
NB = NCOL // B
LOG_M = 15


H = 512
NCH = M // H
LOG_H = 9


def _cmpx_chunk(v, s, K, flip):
    ncols = v.shape[1]
    R = H // (2 * s)
    vr = v.reshape(R, 2, s, ncols)
    a = vr[:, 0]
    b = vr[:, 1]
    mn = jnp.minimum(a, b)
    mx = jnp.maximum(a, b)
    kb = K // (2 * s)
    rio = lax.broadcasted_iota(jnp.int32, (R, 1, 1), 0)
    asc = (rio & kb) == 0
    if flip is not False:
        asc = jnp.logical_xor(asc, flip)
    lo = jnp.where(asc, mn, mx)
    hi = jnp.where(asc, mx, mn)
    return jnp.stack([lo, hi], axis=1).reshape(H, ncols)


def _sort_vbuf(vbuf):

    def chunk_body(c, carry):
        v = vbuf[pl.ds(c * H, H), :]
        flip9 = (c & 1) != 0
        for k in range(1, LOG_H + 1):
            K = 1 << k
            for j in range(k - 1, -1, -1):
                v = _cmpx_chunk(v, 1 << j, K,
                                flip9 if k == LOG_H else False)
        vbuf[pl.ds(c * H, H), :] = v
        return carry

    lax.fori_loop(0, NCH, chunk_body, 0)

    for k in range(LOG_H + 1, LOG_M + 1):
        K = 1 << k
        for j in range(k - 1, LOG_H - 1, -1):
            s = 1 << j
            t = s // H
            jt = t.bit_length() - 1

            def pair_body(r, carry, jt=jt, t=t, s=s, K=K):
                g = r >> jt
                w = r & (t - 1)
                i0 = (g * 2 * t + w) * H
                i1 = i0 + s
                asc = (i0 & K) == 0
                a = vbuf[pl.ds(i0, H), :]
                b = vbuf[pl.ds(i1, H), :]
                mn = jnp.minimum(a, b)
                mx = jnp.maximum(a, b)
                vbuf[pl.ds(i0, H), :] = jnp.where(asc, mn, mx)
                vbuf[pl.ds(i1, H), :] = jnp.where(asc, mx, mn)
                return carry

            lax.fori_loop(0, M // (2 * H), pair_body, 0)

        def intra_body(c, carry, K=K):
            flip = ((c * H) & K) != 0
            v = vbuf[pl.ds(c * H, H), :]
            for j in range(LOG_H - 1, -1, -1):
                v = _cmpx_chunk(v, 1 << j, H, flip)
            vbuf[pl.ds(c * H, H), :] = v
            return carry

        lax.fori_loop(0, NCH, intra_body, 0)


def kernel(x):
    def body(x_ref, out_ref, y_ref, vbuf,
             send_a, recv_a, send_c, recv_c, copy_sem):
        my = lax.axis_index("i")

        barrier = pltpu.get_barrier_semaphore()
        for o in (1, 2, 3):
            pl.semaphore_signal(
                barrier, inc=1,
                device_id=((my + o) % N_DEV,),
                device_id_type=pl.DeviceIdType.MESH,
            )
        pl.semaphore_wait(barrier, 3)

        local_cp = pltpu.make_async_copy(
            x_ref.at[:, pl.ds(my * NCOL, NCOL)],
            y_ref.at[pl.ds(my * M_LOC, M_LOC), :],
            copy_sem,
        )
        local_cp.start()
        sends_a = []
        for o in (1, 2, 3):
            d = (my + o) % N_DEV
            r = pltpu.make_async_remote_copy(
                src_ref=x_ref.at[:, pl.ds(d * NCOL, NCOL)],
                dst_ref=y_ref.at[pl.ds(my * M_LOC, M_LOC), :],
                send_sem=send_a.at[o - 1],
                recv_sem=recv_a.at[my],
                device_id=(d,),
                device_id_type=pl.DeviceIdType.MESH,
            )
            r.start()
            sends_a.append(r)
        local_cp.wait()
        for r in sends_a:
            r.wait_send()
        for o in (1, 2, 3):
            src = (my - o) % N_DEV
            rr = pltpu.make_async_remote_copy(
                src_ref=x_ref.at[:, pl.ds(src * NCOL, NCOL)],
                dst_ref=y_ref.at[pl.ds(src * M_LOC, M_LOC), :],
                send_sem=send_a.at[o - 1],
                recv_sem=recv_a.at[src],
                device_id=(src,),
                device_id_type=pl.DeviceIdType.MESH,
            )
            rr.wait_recv()

        def block_body(b, carry):
            cp_in = pltpu.make_async_copy(
                y_ref.at[:, pl.ds(b * B, B)], vbuf, copy_sem)
            cp_in.start()
            cp_in.wait()
            _sort_vbuf(vbuf)

            col0 = my * NCOL + b * B
            sends_c = []
            for o in (1, 2, 3):
                d = (my + o) % N_DEV
                r = pltpu.make_async_remote_copy(
                    src_ref=vbuf.at[pl.ds(d * M_LOC, M_LOC), :],
                    dst_ref=out_ref.at[:, pl.ds(col0, B)],
                    send_sem=send_c.at[o - 1],
                    recv_sem=recv_c.at[my, b],
                    device_id=(d,),
                    device_id_type=pl.DeviceIdType.MESH,
                )
                r.start()
                sends_c.append(r)
            lc = pltpu.make_async_copy(
                vbuf.at[pl.ds(my * M_LOC, M_LOC), :],
                out_ref.at[:, pl.ds(col0, B)],
                copy_sem,
            )
            lc.start()
            lc.wait()
            for r in sends_c:
                r.wait_send()
            return carry

        lax.fori_loop(0, NB, block_body, 0)

        for o in (1, 2, 3):
            src = (my - o) % N_DEV
            for b in range(NB):
                rr = pltpu.make_async_remote_copy(
                    src_ref=vbuf.at[pl.ds(0, M_LOC), :],
                    dst_ref=out_ref.at[:, pl.ds(src * NCOL + b * B, B)],
                    send_sem=send_c.at[o - 1],
                    recv_sem=recv_c.at[src, b],
                    device_id=(src,),
                    device_id_type=pl.DeviceIdType.MESH,
                )
                rr.wait_recv()

    out, _ = pl.pallas_call(
        body,
        out_shape=[
            jax.ShapeDtypeStruct((M_LOC, N), jnp.float32),
            jax.ShapeDtypeStruct((M, NCOL), jnp.float32),
        ],
        in_specs=[pl.BlockSpec(memory_space=pltpu.HBM)],
        out_specs=[
            pl.BlockSpec(memory_space=pltpu.HBM),
            pl.BlockSpec(memory_space=pltpu.HBM),
        ],
        scratch_shapes=[
            pltpu.VMEM((M, B), jnp.float32),
            pltpu.SemaphoreType.DMA((3,)),
            pltpu.SemaphoreType.DMA((N_DEV,)),
            pltpu.SemaphoreType.DMA((3,)),
            pltpu.SemaphoreType.DMA((N_DEV, NB)),
            pltpu.SemaphoreType.DMA,
        ],
        compiler_params=pltpu.CompilerParams(
            collective_id=0,
            vmem_limit_bytes=64 * 1024 * 1024,
        ),
    )(x)
    return out


# device time: 1393614 ns/iter; 1.5864x vs baseline; 1.5864x over previous
import jax

jax.config.update("jax_compilation_cache_dir", "/tmp/scband_jax_cache")
jax.config.update("jax_persistent_cache_min_compile_time_secs", 0.0)
jax.config.update("jax_persistent_cache_min_entry_size_bytes", 0)

import jax.numpy as jnp
from jax import lax
from jax.experimental import pallas as pl
from jax.experimental.pallas import tpu as pltpu

N_DEV = 4
M_LOC = 8192
N = 1024
M = N_DEV * M_LOC
NCOL = N // N_DEV
B = 128
NB = NCOL // B
LOG_M = 15
H = 512
NCH = M // H
LOG_H = 9


def _cmpx_asc(v, s):
    ncols = v.shape[1]
    R = H // (2 * s)
    vr = v.reshape(R, 2, s, ncols)
    mn = jnp.minimum(vr[:, 0], vr[:, 1])
    mx = jnp.maximum(vr[:, 0], vr[:, 1])
    return jnp.stack([mn, mx], axis=1).reshape(H, ncols)


def _sort_vbuf(vbuf):
    tio = lax.broadcasted_iota(jnp.int32, (H, 1), 0)

    def chunk_body(c, carry):
        v = vbuf[pl.ds(c * H, H), :]
        for k in range(1, LOG_H + 1):
            K = 1 << k
            if k < LOG_H:
                mask = (tio & K) != 0
                v = jnp.where(mask, -v, v)
            else:
                flip9 = (c & 1) != 0
                v = jnp.where(flip9, -v, v)
            for j in range(k - 1, -1, -1):
                v = _cmpx_asc(v, 1 << j)
            if k < LOG_H:
                v = jnp.where(mask, -v, v)
        f9 = (c & 1) != 0
        f10 = ((c * H) & 1024) != 0
        v = jnp.where(f9 != f10, -v, v)
        vbuf[pl.ds(c * H, H), :] = v
        return carry

    lax.fori_loop(0, NCH, chunk_body, 0)

    for k in range(LOG_H + 1, LOG_M + 1):
        K = 1 << k
        for j in range(k - 1, LOG_H - 1, -1):
            s = 1 << j
            t = s // H
            jt = t.bit_length() - 1

            def pair_body(r, carry, jt=jt, t=t, s=s):
                g = r >> jt
                w = r & (t - 1)
                i0 = (g * 2 * t + w) * H
                i1 = i0 + s
                a = vbuf[pl.ds(i0, H), :]
                b = vbuf[pl.ds(i1, H), :]
                vbuf[pl.ds(i0, H), :] = jnp.minimum(a, b)
                vbuf[pl.ds(i1, H), :] = jnp.maximum(a, b)
                return carry

            lax.fori_loop(0, M // (2 * H), pair_body, 0)

        def intra_body(c, carry, k=k, K=K):
            v = vbuf[pl.ds(c * H, H), :]
            for j in range(LOG_H - 1, -1, -1):
                v = _cmpx_asc(v, 1 << j)
            if k < LOG_M:
                fk = ((c * H) & K) != 0
                fk1 = ((c * H) & (2 * K)) != 0
                v = jnp.where(fk != fk1, -v, v)
            vbuf[pl.ds(c * H, H), :] = v
            return carry

        lax.fori_loop(0, NCH, intra_body, 0)


def kernel(x):
    def body(x_ref, out_ref, vbuf0, vbuf1,
             send_a, recv_a, send_c, recv_c, cp_sems, lc_sems):
        my = lax.axis_index("i")
        vbufs = [vbuf0, vbuf1]

        barrier = pltpu.get_barrier_semaphore()
        for o in (1, 2, 3):
            pl.semaphore_signal(
                barrier, inc=1,
                device_id=((my + o) % N_DEV,),
                device_id_type=pl.DeviceIdType.MESH,
            )
        pl.semaphore_wait(barrier, 3)

        sends_a = []
        cps = []
        for b in range(NB):
            for o in (1, 2, 3):
                d = (my + o) % N_DEV
                r = pltpu.make_async_remote_copy(
                    src_ref=x_ref.at[:, pl.ds(d * NCOL + b * B, B)],
                    dst_ref=vbufs[b].at[pl.ds(my * M_LOC, M_LOC), :],
                    send_sem=send_a.at[o - 1, b],
                    recv_sem=recv_a.at[my, b],
                    device_id=(d,),
                    device_id_type=pl.DeviceIdType.MESH,
                )
                r.start()
                sends_a.append(r)
        for b in range(NB):
            cp = pltpu.make_async_copy(
                x_ref.at[:, pl.ds(my * NCOL + b * B, B)],
                vbufs[b].at[pl.ds(my * M_LOC, M_LOC), :],
                cp_sems.at[b],
            )
            cp.start()
            cps.append(cp)

        sends_c = []
        lcs = []
        for b in range(NB):
            cps[b].wait()
            for o in (1, 2, 3):
                src = (my - o) % N_DEV
                rr = pltpu.make_async_remote_copy(
                    src_ref=x_ref.at[:, pl.ds(src * NCOL + b * B, B)],
                    dst_ref=vbufs[b].at[pl.ds(src * M_LOC, M_LOC), :],
                    send_sem=send_a.at[o - 1, b],
                    recv_sem=recv_a.at[src, b],
                    device_id=(src,),
                    device_id_type=pl.DeviceIdType.MESH,
                )
                rr.wait_recv()
            _sort_vbuf(vbufs[b])

            col0 = my * NCOL + b * B
            for o in (1, 2, 3):
                d = (my + o) % N_DEV
                r = pltpu.make_async_remote_copy(
                    src_ref=vbufs[b].at[pl.ds(d * M_LOC, M_LOC), :],
                    dst_ref=out_ref.at[:, pl.ds(col0, B)],
                    send_sem=send_c.at[o - 1, b],
                    recv_sem=recv_c.at[my, b],
                    device_id=(d,),
                    device_id_type=pl.DeviceIdType.MESH,
                )
                r.start()
                sends_c.append(r)
            lc = pltpu.make_async_copy(
                vbufs[b].at[pl.ds(my * M_LOC, M_LOC), :],
                out_ref.at[:, pl.ds(col0, B)],
                lc_sems.at[b],
            )
            lc.start()
            lcs.append(lc)

        for r in sends_a:
            r.wait_send()
        for r in sends_c:
            r.wait_send()
        for lc in lcs:
            lc.wait()
        for o in (1, 2, 3):
            src = (my - o) % N_DEV
            for b in range(NB):
                rr = pltpu.make_async_remote_copy(
                    src_ref=vbufs[b].at[pl.ds(0, M_LOC), :],
                    dst_ref=out_ref.at[:, pl.ds(src * NCOL + b * B, B)],
                    send_sem=send_c.at[o - 1, b],
                    recv_sem=recv_c.at[src, b],
                    device_id=(src,),
                    device_id_type=pl.DeviceIdType.MESH,
                )
                rr.wait_recv()

    return pl.pallas_call(
        body,
        out_shape=jax.ShapeDtypeStruct((M_LOC, N), jnp.float32),
        in_specs=[pl.BlockSpec(memory_space=pltpu.HBM)],
        out_specs=pl.BlockSpec(memory_space=pltpu.HBM),
        scratch_shapes=[
            pltpu.VMEM((M, B), jnp.float32),
            pltpu.VMEM((M, B), jnp.float32),
            pltpu.SemaphoreType.DMA((3, NB)),
            pltpu.SemaphoreType.DMA((N_DEV, NB)),
            pltpu.SemaphoreType.DMA((3, NB)),
            pltpu.SemaphoreType.DMA((N_DEV, NB)),
            pltpu.SemaphoreType.DMA((NB,)),
            pltpu.SemaphoreType.DMA((NB,)),
        ],
        compiler_params=pltpu.CompilerParams(
            collective_id=0,
            vmem_limit_bytes=64 * 1024 * 1024,
        ),
    )(x)
